# baseline (speedup 1.0000x reference)
import jax
import jax.numpy as jnp
from jax import lax
from jax.experimental import pallas as pl
from jax.experimental.pallas import tpu as pltpu

_D = 2048
_E = 16
_TOPK = 2
_T = 16384
_BT = 512


def _router_body(h_ref, wt_ref, b_ref, logits_ref, wts_ref, sel_ref, mask_ref):
    h = h_ref[...]
    w = wt_ref[...]
    lt = lax.dot_general(w, h, (((1,), (1,)), ((), ())),
                         preferred_element_type=jnp.float32) + b_ref[...]
    logits_ref[...] = lt

    e_iota = lax.broadcasted_iota(jnp.int32, (_E, _BT), 0)
    v1 = jnp.max(lt, axis=0, keepdims=True)
    i1 = jnp.min(jnp.where(lt == v1, e_iota, _E), axis=0, keepdims=True)
    l2 = jnp.where(e_iota == i1, jnp.float32(-jnp.inf), lt)
    v2 = jnp.max(l2, axis=0, keepdims=True)
    i2 = jnp.min(jnp.where(l2 == v2, e_iota, _E), axis=0, keepdims=True)

    e2 = jnp.exp(v2 - v1)
    denom = 1.0 + e2
    wts_ref[...] = jnp.concatenate([1.0 / denom, e2 / denom], axis=0)
    sel_ref[...] = jnp.concatenate([i1, i2], axis=0)

    r_iota = lax.broadcasted_iota(jnp.int32, (_E * _TOPK, _BT), 0)
    sel_r = jnp.where((r_iota & 1) == 0, i1, i2)
    mask_ref[...] = (sel_r == (r_iota >> 1)).astype(jnp.int32)


def kernel(hidden_states, W, b):
    b2 = b.reshape(_E, 1)
    grid = (_T // _BT,)
    logits_t, wts_t, sel_t, mask_t = pl.pallas_call(
        _router_body,
        grid=grid,
        in_specs=[
            pl.BlockSpec((_BT, _D), lambda i: (i, 0)),
            pl.BlockSpec((_E, _D), lambda i: (0, 0)),
            pl.BlockSpec((_E, 1), lambda i: (0, 0)),
        ],
        out_specs=[
            pl.BlockSpec((_E, _BT), lambda i: (0, i)),
            pl.BlockSpec((_TOPK, _BT), lambda i: (0, i)),
            pl.BlockSpec((_TOPK, _BT), lambda i: (0, i)),
            pl.BlockSpec((_E * _TOPK, _BT), lambda i: (0, i)),
        ],
        out_shape=[
            jax.ShapeDtypeStruct((_E, _T), jnp.float32),
            jax.ShapeDtypeStruct((_TOPK, _T), jnp.float32),
            jax.ShapeDtypeStruct((_TOPK, _T), jnp.int32),
            jax.ShapeDtypeStruct((_E * _TOPK, _T), jnp.int32),
        ],
        compiler_params=pltpu.CompilerParams(
            dimension_semantics=("parallel",),
        ),
    )(hidden_states, W, b2)
    return (logits_t.T, wts_t.T, sel_t.T, mask_t.reshape(_E, _TOPK, _T))


# speedup vs baseline: 1.1765x; 1.1765x over previous
import jax
import jax.numpy as jnp
from jax import lax
from jax.experimental import pallas as pl
from jax.experimental.pallas import tpu as pltpu

_D = 2048
_E = 16
_TOPK = 2
_T = 16384
_BT = 1024


def _router_body(h_ref, wt_ref, b_ref, logits_ref, wts_ref, sel_ref, mask_ref):
    h = h_ref[...]
    w = wt_ref[...]
    lt = lax.dot_general(w, h, (((1,), (1,)), ((), ())),
                         preferred_element_type=jnp.float32) + b_ref[...]
    logits_ref[...] = lt

    e_iota = lax.broadcasted_iota(jnp.int32, (_E, _BT), 0)
    v1 = jnp.max(lt, axis=0, keepdims=True)
    i1 = jnp.min(jnp.where(lt == v1, e_iota, _E), axis=0, keepdims=True)
    l2 = jnp.where(e_iota == i1, jnp.float32(-jnp.inf), lt)
    v2 = jnp.max(l2, axis=0, keepdims=True)
    i2 = jnp.min(jnp.where(l2 == v2, e_iota, _E), axis=0, keepdims=True)

    e2 = jnp.exp(v2 - v1)
    denom = 1.0 + e2
    wts_ref[...] = jnp.concatenate([1.0 / denom, e2 / denom], axis=0)
    sel_ref[...] = jnp.concatenate([i1, i2], axis=0)

    r_iota = lax.broadcasted_iota(jnp.int32, (_E * _TOPK, _BT), 0)
    sel_r = jnp.where((r_iota & 1) == 0, i1, i2)
    mask_ref[...] = (sel_r == (r_iota >> 1)).astype(jnp.int32)


def kernel(hidden_states, W, b):
    b2 = b.reshape(_E, 1)
    grid = (_T // _BT,)
    logits_t, wts_t, sel_t, mask_t = pl.pallas_call(
        _router_body,
        grid=grid,
        in_specs=[
            pl.BlockSpec((_BT, _D), lambda i: (i, 0)),
            pl.BlockSpec((_E, _D), lambda i: (0, 0)),
            pl.BlockSpec((_E, 1), lambda i: (0, 0)),
        ],
        out_specs=[
            pl.BlockSpec((_E, _BT), lambda i: (0, i)),
            pl.BlockSpec((_TOPK, _BT), lambda i: (0, i)),
            pl.BlockSpec((_TOPK, _BT), lambda i: (0, i)),
            pl.BlockSpec((_E * _TOPK, _BT), lambda i: (0, i)),
        ],
        out_shape=[
            jax.ShapeDtypeStruct((_E, _T), jnp.float32),
            jax.ShapeDtypeStruct((_TOPK, _T), jnp.float32),
            jax.ShapeDtypeStruct((_TOPK, _T), jnp.int32),
            jax.ShapeDtypeStruct((_E * _TOPK, _T), jnp.int32),
        ],
        compiler_params=pltpu.CompilerParams(
            dimension_semantics=("parallel",),
        ),
    )(hidden_states, W, b2)
    return (logits_t.T, wts_t.T, sel_t.T, mask_t.reshape(_E, _TOPK, _T))
